# baseline (speedup 1.0000x reference)
import functools

import jax
import jax.numpy as jnp
from jax import lax
from jax.experimental import pallas as pl
from jax.experimental.pallas import tpu as pltpu
from jax.experimental.pallas import tpu_sc as plsc

_FIELD0_ROWS = 1_000_000
_BATCH = 16384
_D = 16

_info = plsc.get_sparse_core_info()
_NC, _NS, _L = _info.num_cores, _info.num_subcores, _info.num_lanes
_NW = _NC * _NS
_BPW = _BATCH // _NW
_CHUNK = 128
_NCHUNK = _BPW // _CHUNK

_GATHER_DNUMS = lax.GatherDimensionNumbers(
    offset_dims=(), collapsed_slice_dims=(0,), start_index_map=(0,))


def _permute(t, idx):
    return lax.gather(t, idx[:, None], _GATHER_DNUMS, (1,),
                      mode=lax.GatherScatterMode.PROMISE_IN_BOUNDS)


def _gmf_body(x0_hbm, x1_hbm, table_hbm, w_hbm, b_hbm, out_hbm,
              idx0, idx1, rows0, rows1, w_v, b_v, out_v, sem):
    wid = lax.axis_index("s") * _NC + lax.axis_index("c")
    base = wid * _BPW

    for c in range(_NCHUNK):
        pltpu.sync_copy(x0_hbm.at[pl.ds(base + c * _CHUNK, _CHUNK)], idx0.at[c])
        pltpu.sync_copy(x1_hbm.at[pl.ds(base + c * _CHUNK, _CHUNK)], idx1.at[c])
    pltpu.sync_copy(w_hbm, w_v)
    pltpu.sync_copy(b_hbm, b_v)

    for c in range(_NCHUNK):
        for k in range(_CHUNK // _L):
            sl = pl.ds(k * _L, _L)
            idx1[c, sl] = idx1[c, sl] + _FIELD0_ROWS

    copies = []
    for c in range(_NCHUNK):
        copies.append(
            pltpu.async_copy(table_hbm.at[idx0.at[c]], rows0.at[c], sem))
        copies.append(
            pltpu.async_copy(table_hbm.at[idx1.at[c]], rows1.at[c], sem))
    for cp in copies:
        cp.wait()

    w = w_v[...]
    bv = b_v[...]
    lane = lax.iota(jnp.int32, _L)

    for c in range(_NCHUNK):
        def block(g, _, c=c):
            ov = bv
            for j in range(_L):
                r = g * _L + j
                t = rows0[c, r] * rows1[c, r] * w
                for st in (8, 4, 2, 1):
                    t = t + _permute(t, lane ^ st)
                ov = jnp.where(lane == j, ov + t, ov)
            out_v[pl.ds((c * (_CHUNK // _L) + g) * _L, _L)] = (
                jnp.maximum(ov, 0.0))
            return 0

        lax.fori_loop(0, _CHUNK // _L, block, 0)

    pltpu.sync_copy(out_v, out_hbm.at[pl.ds(base, _BPW)])


@functools.partial(jax.jit, static_argnames=())
def _gmf(x0, x1, table, w, bvec):
    mesh = plsc.VectorSubcoreMesh(core_axis_name="c", subcore_axis_name="s")
    run = functools.partial(
        pl.kernel,
        mesh=mesh,
        compiler_params=pltpu.CompilerParams(use_tc_tiling_on_sc=False),
        out_type=jax.ShapeDtypeStruct((_BATCH,), jnp.float32),
        scratch_types=[
            pltpu.VMEM((_NCHUNK, _CHUNK), jnp.int32),
            pltpu.VMEM((_NCHUNK, _CHUNK), jnp.int32),
            pltpu.VMEM((_NCHUNK, _CHUNK, _D), jnp.float32),
            pltpu.VMEM((_NCHUNK, _CHUNK, _D), jnp.float32),
            pltpu.VMEM((_D,), jnp.float32),
            pltpu.VMEM((_L,), jnp.float32),
            pltpu.VMEM((_BPW,), jnp.float32),
            pltpu.SemaphoreType.DMA,
        ],
    )(_gmf_body)
    return run(x0, x1, table, w, bvec)


def kernel(x, table, W, b):
    x0 = x[:, 0].astype(jnp.int32)
    x1 = x[:, 1].astype(jnp.int32)
    w = W.reshape(_D)
    bvec = jnp.broadcast_to(b.reshape(()), (_L,)).astype(jnp.float32)
    y = _gmf(x0, x1, table, w, bvec)
    return y.reshape(_BATCH, 1)
